# baseline (speedup 1.0000x reference)
import functools

import jax
import jax.numpy as jnp
from jax import lax
from jax.experimental import pallas as pl
from jax.experimental.pallas import tpu as pltpu
from jax.experimental.pallas import tpu_sc as plsc
from jax.experimental import layout as jlayout

BATCH = 16384
FIELDS = 26
NUM_CORES = 2
NUM_SUBCORES = 16
LANES = 16
NW = NUM_CORES * NUM_SUBCORES
ROWS_W = BATCH // NW
ELEMS_W = ROWS_W * FIELDS
IDX_MINOR = 128
IDX_MAJOR = ELEMS_W // IDX_MINOR
ROW_CHUNKS = ROWS_W // LANES
HALF_R = ROWS_W // 2
HALF_E = HALF_R * FIELDS


def _lr_body(idx_hbm, val_hbm, table_hbm, bias_hbm, out_hbm,
             idx_v, w_v, val_v, bias_v, y_v, sem, sem2, sem3):
    wid = lax.axis_index("s") * NUM_CORES + lax.axis_index("c")

    tbl = table_hbm.at[0]
    pltpu.sync_copy(idx_hbm.at[wid, pl.ds(0, HALF_E)],
                    idx_v.at[pl.ds(0, HALF_E)])
    g0 = pltpu.async_copy(tbl.at[idx_v.at[pl.ds(0, HALF_E)]],
                          w_v.at[pl.ds(0, HALF_E)], sem)
    pltpu.sync_copy(idx_hbm.at[wid, pl.ds(HALF_E, HALF_E)],
                    idx_v.at[pl.ds(HALF_E, HALF_E)])
    g1 = pltpu.async_copy(tbl.at[idx_v.at[pl.ds(HALF_E, HALF_E)]],
                          w_v.at[pl.ds(HALF_E, HALF_E)], sem2)
    val_cp = pltpu.async_copy(val_hbm.at[wid], val_v, sem3)
    pltpu.sync_copy(bias_hbm, bias_v)
    val_cp.wait()

    bias16 = bias_v[...]

    def make_red(h):
        def red_body(c, carry):
            acc = jnp.zeros((LANES,), jnp.float32)
            for f in range(FIELDS):
                off = h * HALF_E + f * HALF_R + c * LANES
                acc = acc + w_v[pl.ds(off, LANES)] * val_v[pl.ds(off, LANES)]
            y = 1.0 / (1.0 + jnp.exp(-(acc + bias16)))
            y_v[pl.ds(h * HALF_R + c * LANES, LANES)] = y
            return carry
        return red_body

    g0.wait()
    lax.fori_loop(0, HALF_R // LANES, make_red(0), 0)
    g1.wait()
    lax.fori_loop(0, HALF_R // LANES, make_red(1), 0)

    pltpu.sync_copy(y_v, out_hbm.at[pl.ds(wid * ROWS_W, ROWS_W)])


@functools.partial(jax.jit, static_argnames=())
def kernel(feat_index, feat_value, weights, bias):
    idx = feat_index.astype(jnp.int32).reshape(NW, 2, HALF_R, FIELDS)
    idx = jnp.swapaxes(idx, 2, 3).reshape(NW, ELEMS_W)
    val = feat_value.reshape(NW, 2, HALF_R, FIELDS)
    val = jnp.swapaxes(val, 2, 3).reshape(NW, ELEMS_W)
    table = jlayout.with_layout_constraint(
        weights.reshape(1, -1),
        jlayout.Layout((1, 0), tiling=((1, 128),)),
    )
    bias16 = jnp.broadcast_to(bias.astype(jnp.float32), (LANES,))

    run = pl.kernel(
        _lr_body,
        out_type=jax.ShapeDtypeStruct((BATCH,), jnp.float32),
        mesh=plsc.VectorSubcoreMesh(core_axis_name="c", subcore_axis_name="s"),
        scratch_types=[
            pltpu.VMEM((ELEMS_W,), jnp.int32),
            pltpu.VMEM((ELEMS_W,), jnp.float32),
            pltpu.VMEM((ELEMS_W,), jnp.float32),
            pltpu.VMEM((LANES,), jnp.float32),
            pltpu.VMEM((ROWS_W,), jnp.float32),
            pltpu.SemaphoreType.DMA,
            pltpu.SemaphoreType.DMA,
            pltpu.SemaphoreType.DMA,
        ],
    )
    return run(idx, val, table, bias16)


# speedup vs baseline: 1.0809x; 1.0809x over previous
import functools

import jax
import jax.numpy as jnp
from jax import lax
from jax.experimental import pallas as pl
from jax.experimental.pallas import tpu as pltpu
from jax.experimental.pallas import tpu_sc as plsc
from jax.experimental import layout as jlayout

BATCH = 16384
FIELDS = 26
NUM_CORES = 2
NUM_SUBCORES = 16
LANES = 16
NW = NUM_CORES * NUM_SUBCORES
ROWS_W = BATCH // NW
ELEMS_W = ROWS_W * FIELDS
IDX_MINOR = 128
IDX_MAJOR = ELEMS_W // IDX_MINOR
ROW_CHUNKS = ROWS_W // LANES


def _lr_body(idx_hbm, val_hbm, table_hbm, bias_hbm, out_hbm,
             idx_v, w_v, val_v, bias_v, y_v, sem, sem2):
    wid = lax.axis_index("s") * NUM_CORES + lax.axis_index("c")

    pltpu.sync_copy(idx_hbm.at[wid], idx_v)

    gather = pltpu.async_copy(table_hbm.at[0].at[idx_v], w_v, sem)
    val_cp = pltpu.async_copy(val_hbm.at[wid], val_v, sem2)
    pltpu.sync_copy(bias_hbm, bias_v)
    val_cp.wait()
    gather.wait()

    bias16 = bias_v[...]

    def red_body(c, carry):
        acc = jnp.zeros((LANES,), jnp.float32)
        for f in range(FIELDS):
            off = f * ROWS_W + c * LANES
            acc = acc + w_v[pl.ds(off, LANES)] * val_v[pl.ds(off, LANES)]
        y = 1.0 / (1.0 + jnp.exp(-(acc + bias16)))
        y_v[pl.ds(c * LANES, LANES)] = y
        return carry

    lax.fori_loop(0, ROW_CHUNKS, red_body, 0)

    pltpu.sync_copy(y_v, out_hbm.at[pl.ds(wid * ROWS_W, ROWS_W)])


@functools.partial(jax.jit, static_argnames=())
def kernel(feat_index, feat_value, weights, bias):
    idx = feat_index.astype(jnp.int32).reshape(NW, ROWS_W, FIELDS)
    idx = jnp.swapaxes(idx, 1, 2).reshape(NW, ELEMS_W)
    val = feat_value.reshape(NW, ROWS_W, FIELDS)
    val = jnp.swapaxes(val, 1, 2).reshape(NW, ELEMS_W)
    table = jlayout.with_layout_constraint(
        weights.reshape(1, -1),
        jlayout.Layout((1, 0), tiling=((1, 128),)),
    )
    bias16 = jnp.broadcast_to(bias.astype(jnp.float32), (LANES,))

    run = pl.kernel(
        _lr_body,
        out_type=jax.ShapeDtypeStruct((BATCH,), jnp.float32),
        mesh=plsc.VectorSubcoreMesh(core_axis_name="c", subcore_axis_name="s"),
        scratch_types=[
            pltpu.VMEM((ELEMS_W,), jnp.int32),
            pltpu.VMEM((ELEMS_W,), jnp.float32),
            pltpu.VMEM((ELEMS_W,), jnp.float32),
            pltpu.VMEM((LANES,), jnp.float32),
            pltpu.VMEM((ROWS_W,), jnp.float32),
            pltpu.SemaphoreType.DMA,
            pltpu.SemaphoreType.DMA,
        ],
    )
    return run(idx, val, table, bias16)
